# baseline (speedup 1.0000x reference)
import functools

import jax
import jax.numpy as jnp
from jax import lax
from jax.experimental import pallas as pl
from jax.experimental.pallas import tpu as pltpu
from jax.experimental.pallas import tpu_sc as plsc

_N = 10000
_E = 320000
_H = 128
_L = 4
_NA = 28

_NC = 2
_NS = 16
_NT = _NC * _NS
_C = 128
_NCH = 80
_NGRP = 10
_EPT = _C * _NCH
_EPAD = _NT * _EPT
_NP = 10240
_RPT = _NP // _NS

_SC_MESH = plsc.VectorSubcoreMesh(core_axis_name="c", subcore_axis_name="s",
                                  num_cores=_NC, num_subcores=_NS)


def _sc_lap_body(src_hbm, dst_hbm, xs_hbm, z128_hbm, out_hbm,
                 src_v, dst_v, rows_v, acc_sh, gsem, dsem, ssem):
    cid = lax.axis_index("c")
    sid = lax.axis_index("s")
    pltpu.sync_copy(z128_hbm, acc_sh.at[pl.ds(sid * _RPT, _RPT)])
    pltpu.sync_copy(src_hbm.at[cid, sid], src_v)
    plsc.subcore_barrier()

    pltpu.async_copy(dst_hbm.at[cid, sid, 0], dst_v.at[0], dsem.at[0])
    pltpu.async_copy(xs_hbm.at[src_v.at[0]], rows_v.at[0], gsem.at[0])

    def gbody(g, carry):
        gslot = lax.rem(g, 2)

        @pl.when(g >= 1)
        def _():
            pltpu.make_async_copy(rows_v.at[1], acc_sh.at[dst_v.at[1 - gslot, 7]],
                                  ssem).wait()

        @pl.when(g + 1 < _NGRP)
        def _():
            pltpu.async_copy(dst_hbm.at[cid, sid, g + 1],
                             dst_v.at[1 - gslot], dsem.at[1 - gslot])

        pltpu.make_async_copy(dst_hbm.at[cid, sid, g], dst_v.at[gslot],
                              dsem.at[gslot]).wait()

        for k in range(8):
            i = g * 8 + k
            rslot = k % 2
            pltpu.make_async_copy(xs_hbm.at[src_v.at[i]], rows_v.at[rslot],
                                  gsem.at[rslot]).wait()
            if k >= 1:
                pltpu.make_async_copy(rows_v.at[1 - rslot],
                                      acc_sh.at[dst_v.at[gslot, k - 1]],
                                      ssem).wait()
            pltpu.async_copy(rows_v.at[rslot], acc_sh.at[dst_v.at[gslot, k]],
                             ssem, add=True)
            if k < 7:
                pltpu.async_copy(xs_hbm.at[src_v.at[i + 1]],
                                 rows_v.at[1 - rslot], gsem.at[1 - rslot])
            else:
                @pl.when(g + 1 < _NGRP)
                def _():
                    pltpu.async_copy(xs_hbm.at[src_v.at[i + 1]],
                                     rows_v.at[1 - rslot], gsem.at[1 - rslot])
        return carry

    lax.fori_loop(0, _NGRP, gbody, 0)
    pltpu.make_async_copy(rows_v.at[1], acc_sh.at[dst_v.at[(_NGRP - 1) % 2, 7]],
                          ssem).wait()
    plsc.subcore_barrier()
    pltpu.sync_copy(acc_sh.at[pl.ds(sid * _RPT, _RPT)],
                    out_hbm.at[cid, pl.ds(sid * _RPT, _RPT)])


def _make_sc_lap(interpret=False):
    return pl.kernel(
        _sc_lap_body,
        out_type=jax.ShapeDtypeStruct((_NC, _NP, _H), jnp.float32),
        mesh=_SC_MESH,
        scratch_types=[
            pltpu.VMEM((_NCH, _C), jnp.int32),
            pltpu.VMEM((2, 8, _C), jnp.int32),
            pltpu.VMEM((2, _C, _H), jnp.float32),
            pltpu.VMEM_SHARED((_NP, _H), jnp.float32),
            pltpu.SemaphoreType.DMA((2,)),
            pltpu.SemaphoreType.DMA((2,)),
            pltpu.SemaphoreType.DMA,
        ],
        interpret=interpret,
    )


_sc_lap = _make_sc_lap()


def _tc_prologue_body(degp_ref, h_ref, emb_ref, norm_ref, x_ref, xs_ref):
    deg = degp_ref[0, :_N, 0:1] + degp_ref[1, :_N, 0:1]
    norm = 1.0 / jnp.sqrt(jnp.maximum(deg, 1.0))
    norm_ref[...] = norm
    oh = (h_ref[...] == lax.broadcasted_iota(jnp.int32, (1, _NA), 1))
    x = jnp.dot(oh.astype(jnp.float32), emb_ref[...],
                preferred_element_type=jnp.float32)
    x_ref[...] = x
    xs_ref[...] = x * norm


def _tc_mid_body(agg_ref, norm_ref, x1_ref, ys_ref):
    n = norm_ref[...]
    x1 = -((agg_ref[0, :_N] + agg_ref[1, :_N]) * n)
    x1_ref[...] = x1
    ys_ref[...] = x1 * n


def _layer_tail(x, x1, agg0, agg1, n, w_ref, b_ref, g_ref, be_ref):
    x2 = -2.0 * ((agg0 + agg1) * n) - x
    out = (jnp.dot(x, w_ref[0], preferred_element_type=jnp.float32)
           + jnp.dot(x1, w_ref[1], preferred_element_type=jnp.float32)
           + jnp.dot(x2, w_ref[2], preferred_element_type=jnp.float32)
           + (b_ref[0] + b_ref[1] + b_ref[2]))
    out = out * jnp.float32(1.0 / 100.0)
    mean = jnp.mean(out, axis=0, keepdims=True)
    cen = out - mean
    var = jnp.mean(cen * cen, axis=0, keepdims=True)
    out = cen / jnp.sqrt(var + 1e-5) * g_ref[...] + be_ref[...]
    out = jnp.maximum(out, 0.0)
    return x + out


def _tc_tail_body(x_ref, x1_ref, agg_ref, norm_ref, w_ref, b_ref,
                  g_ref, be_ref, xn_ref, xsn_ref):
    n = norm_ref[...]
    xn = _layer_tail(x_ref[...], x1_ref[...], agg_ref[0, :_N], agg_ref[1, :_N],
                     n, w_ref, b_ref, g_ref, be_ref)
    xn_ref[...] = xn
    xsn_ref[...] = xn * n


def _tc_tail_last_body(x_ref, x1_ref, agg_ref, norm_ref, w_ref, b_ref,
                       g_ref, be_ref, w1_ref, b1_ref, w2_ref, b2_ref,
                       w3_ref, b3_ref, y_ref):
    n = norm_ref[...]
    xn = _layer_tail(x_ref[...], x1_ref[...], agg_ref[0, :_N], agg_ref[1, :_N],
                     n, w_ref, b_ref, g_ref, be_ref)
    hg = jnp.mean(xn, axis=0, keepdims=True)
    y = jnp.maximum(jnp.dot(hg, w1_ref[...],
                            preferred_element_type=jnp.float32) + b1_ref[...], 0.0)
    y = jnp.maximum(jnp.dot(y, w2_ref[...],
                            preferred_element_type=jnp.float32) + b2_ref[...], 0.0)
    y_ref[...] = jnp.dot(y, w3_ref[...],
                         preferred_element_type=jnp.float32) + b3_ref[...]


def _f32(*shape):
    return jax.ShapeDtypeStruct(shape, jnp.float32)


_tc_prologue = pl.pallas_call(
    _tc_prologue_body, out_shape=[_f32(_N, 1), _f32(_N, _H), _f32(_N, _H)])
_tc_mid = pl.pallas_call(
    _tc_mid_body, out_shape=[_f32(_N, _H), _f32(_N, _H)])
_tc_tail = pl.pallas_call(
    _tc_tail_body, out_shape=[_f32(_N, _H), _f32(_N, _H)])
_tc_tail_last = pl.pallas_call(
    _tc_tail_last_body, out_shape=_f32(1, 1))


def kernel(edge_index, h, e, emb, W, b, bn_gamma, bn_beta,
           mlp_W1, mlp_b1, mlp_W2, mlp_b2, mlp_W3, mlp_b3):
    pad = _EPAD - _E
    pad_src = (jnp.arange(pad, dtype=jnp.int32) * 131) % _N
    pad_dst = _N + (jnp.arange(pad, dtype=jnp.int32) % (_NP - _N))
    src3 = jnp.concatenate([edge_index[0], pad_src]).reshape(
        _NC, _NS, _NCH, _C)
    dst4 = jnp.concatenate([edge_index[1], pad_dst]).reshape(
        _NC, _NS, _NGRP, 8, _C)
    z128 = jnp.zeros((_RPT, _H), jnp.float32)

    onesN = jnp.ones((_N, _H), jnp.float32)
    degp = _sc_lap(src3, dst4, onesN, z128)
    norm, x, xs = _tc_prologue(degp, h.reshape(_N, 1), emb)

    for l in range(_L):
        agg1 = _sc_lap(src3, dst4, xs, z128)
        x1, ys = _tc_mid(agg1, norm)
        agg2 = _sc_lap(src3, dst4, ys, z128)
        bl = b[l][:, None, :]
        gl = bn_gamma[l][None, :]
        bel = bn_beta[l][None, :]
        if l < _L - 1:
            x, xs = _tc_tail(x, x1, agg2, norm, W[l], bl, gl, bel)
        else:
            y = _tc_tail_last(x, x1, agg2, norm, W[l], bl, gl, bel,
                              mlp_W1, mlp_b1[None, :], mlp_W2, mlp_b2[None, :],
                              mlp_W3, mlp_b3[None, :])
    return y


# speedup vs baseline: 1.1837x; 1.1837x over previous
import functools

import jax
import jax.numpy as jnp
from jax import lax
from jax.experimental import pallas as pl
from jax.experimental.pallas import tpu as pltpu
from jax.experimental.pallas import tpu_sc as plsc

_N = 10000
_E = 320000
_H = 128
_L = 4
_NA = 28

_NC = 2
_NS = 16
_NT = _NC * _NS
_C = 128
_NCH = 80
_NGRP = 10
_EPT = _C * _NCH
_EPAD = _NT * _EPT
_NP = 10240
_RPT = _NP // _NS

_SC_MESH = plsc.VectorSubcoreMesh(core_axis_name="c", subcore_axis_name="s",
                                  num_cores=_NC, num_subcores=_NS)


def _sc_lap_body(src_hbm, dst_hbm, xs_hbm, z128_hbm, out_hbm,
                 src_v, dst_v, rows_v, acc_sh, gsem, dsem):
    cid = lax.axis_index("c")
    sid = lax.axis_index("s")
    pltpu.sync_copy(z128_hbm, acc_sh.at[pl.ds(sid * _RPT, _RPT)])
    pltpu.sync_copy(src_hbm.at[cid, sid], src_v)
    plsc.subcore_barrier()

    pltpu.async_copy(dst_hbm.at[cid, sid, 0], dst_v.at[0], dsem.at[0])
    pltpu.async_copy(xs_hbm.at[src_v.at[0]], rows_v.at[0], gsem.at[0])

    def gbody(g, carry):
        gslot = lax.rem(g, 2)

        @pl.when(g + 1 < _NGRP)
        def _():
            pltpu.async_copy(dst_hbm.at[cid, sid, g + 1],
                             dst_v.at[1 - gslot], dsem.at[1 - gslot])

        pltpu.make_async_copy(dst_hbm.at[cid, sid, g], dst_v.at[gslot],
                              dsem.at[gslot]).wait()

        for k in range(8):
            i = g * 8 + k
            rslot = k % 2
            if k < 7:
                pltpu.async_copy(xs_hbm.at[src_v.at[i + 1]],
                                 rows_v.at[1 - rslot], gsem.at[1 - rslot])
            else:
                @pl.when(g + 1 < _NGRP)
                def _():
                    pltpu.async_copy(xs_hbm.at[src_v.at[i + 1]],
                                     rows_v.at[1 - rslot], gsem.at[1 - rslot])
            pltpu.make_async_copy(xs_hbm.at[src_v.at[i]], rows_v.at[rslot],
                                  gsem.at[rslot]).wait()
            pltpu.sync_copy(rows_v.at[rslot], acc_sh.at[dst_v.at[gslot, k]],
                            add=True)
        return carry

    lax.fori_loop(0, _NGRP, gbody, 0)
    plsc.subcore_barrier()
    pltpu.sync_copy(acc_sh.at[pl.ds(sid * _RPT, _RPT)],
                    out_hbm.at[cid, pl.ds(sid * _RPT, _RPT)])


def _make_sc_lap(interpret=False):
    return pl.kernel(
        _sc_lap_body,
        out_type=jax.ShapeDtypeStruct((_NC, _NP, _H), jnp.float32),
        mesh=_SC_MESH,
        scratch_types=[
            pltpu.VMEM((_NCH, _C), jnp.int32),
            pltpu.VMEM((2, 8, _C), jnp.int32),
            pltpu.VMEM((2, _C, _H), jnp.float32),
            pltpu.VMEM_SHARED((_NP, _H), jnp.float32),
            pltpu.SemaphoreType.DMA((2,)),
            pltpu.SemaphoreType.DMA((2,)),
        ],
        interpret=interpret,
    )


_sc_lap = _make_sc_lap()


def _tc_prologue_body(degp_ref, h_ref, emb_ref, norm_ref, x_ref, xs_ref):
    deg = degp_ref[0, :_N, 0:1] + degp_ref[1, :_N, 0:1]
    norm = 1.0 / jnp.sqrt(jnp.maximum(deg, 1.0))
    norm_ref[...] = norm
    oh = (h_ref[...] == lax.broadcasted_iota(jnp.int32, (1, _NA), 1))
    x = jnp.dot(oh.astype(jnp.float32), emb_ref[...],
                preferred_element_type=jnp.float32)
    x_ref[...] = x
    xs_ref[...] = x * norm


def _tc_mid_body(agg_ref, norm_ref, x1_ref, ys_ref):
    n = norm_ref[...]
    x1 = -((agg_ref[0, :_N] + agg_ref[1, :_N]) * n)
    x1_ref[...] = x1
    ys_ref[...] = x1 * n


def _layer_tail(x, x1, agg0, agg1, n, w_ref, b_ref, g_ref, be_ref):
    x2 = -2.0 * ((agg0 + agg1) * n) - x
    out = (jnp.dot(x, w_ref[0], preferred_element_type=jnp.float32)
           + jnp.dot(x1, w_ref[1], preferred_element_type=jnp.float32)
           + jnp.dot(x2, w_ref[2], preferred_element_type=jnp.float32)
           + (b_ref[0] + b_ref[1] + b_ref[2]))
    out = out * jnp.float32(1.0 / 100.0)
    mean = jnp.mean(out, axis=0, keepdims=True)
    cen = out - mean
    var = jnp.mean(cen * cen, axis=0, keepdims=True)
    out = cen / jnp.sqrt(var + 1e-5) * g_ref[...] + be_ref[...]
    out = jnp.maximum(out, 0.0)
    return x + out


def _tc_tail_body(x_ref, x1_ref, agg_ref, norm_ref, w_ref, b_ref,
                  g_ref, be_ref, xn_ref, xsn_ref):
    n = norm_ref[...]
    xn = _layer_tail(x_ref[...], x1_ref[...], agg_ref[0, :_N], agg_ref[1, :_N],
                     n, w_ref, b_ref, g_ref, be_ref)
    xn_ref[...] = xn
    xsn_ref[...] = xn * n


def _tc_tail_last_body(x_ref, x1_ref, agg_ref, norm_ref, w_ref, b_ref,
                       g_ref, be_ref, w1_ref, b1_ref, w2_ref, b2_ref,
                       w3_ref, b3_ref, y_ref):
    n = norm_ref[...]
    xn = _layer_tail(x_ref[...], x1_ref[...], agg_ref[0, :_N], agg_ref[1, :_N],
                     n, w_ref, b_ref, g_ref, be_ref)
    hg = jnp.mean(xn, axis=0, keepdims=True)
    y = jnp.maximum(jnp.dot(hg, w1_ref[...],
                            preferred_element_type=jnp.float32) + b1_ref[...], 0.0)
    y = jnp.maximum(jnp.dot(y, w2_ref[...],
                            preferred_element_type=jnp.float32) + b2_ref[...], 0.0)
    y_ref[...] = jnp.dot(y, w3_ref[...],
                         preferred_element_type=jnp.float32) + b3_ref[...]


def _f32(*shape):
    return jax.ShapeDtypeStruct(shape, jnp.float32)


_tc_prologue = pl.pallas_call(
    _tc_prologue_body, out_shape=[_f32(_N, 1), _f32(_N, _H), _f32(_N, _H)])
_tc_mid = pl.pallas_call(
    _tc_mid_body, out_shape=[_f32(_N, _H), _f32(_N, _H)])
_tc_tail = pl.pallas_call(
    _tc_tail_body, out_shape=[_f32(_N, _H), _f32(_N, _H)])
_tc_tail_last = pl.pallas_call(
    _tc_tail_last_body, out_shape=_f32(1, 1))


def kernel(edge_index, h, e, emb, W, b, bn_gamma, bn_beta,
           mlp_W1, mlp_b1, mlp_W2, mlp_b2, mlp_W3, mlp_b3):
    pad = _EPAD - _E
    pad_src = (jnp.arange(pad, dtype=jnp.int32) * 131) % _N
    pad_dst = _N + (jnp.arange(pad, dtype=jnp.int32) % (_NP - _N))
    src3 = jnp.concatenate([edge_index[0], pad_src]).reshape(
        _NC, _NS, _NCH, _C)
    dst4 = jnp.concatenate([edge_index[1], pad_dst]).reshape(
        _NC, _NS, _NGRP, 8, _C)
    z128 = jnp.zeros((_RPT, _H), jnp.float32)

    onesN = jnp.ones((_N, _H), jnp.float32)
    degp = _sc_lap(src3, dst4, onesN, z128)
    norm, x, xs = _tc_prologue(degp, h.reshape(_N, 1), emb)

    for l in range(_L):
        agg1 = _sc_lap(src3, dst4, xs, z128)
        x1, ys = _tc_mid(agg1, norm)
        agg2 = _sc_lap(src3, dst4, ys, z128)
        bl = b[l][:, None, :]
        gl = bn_gamma[l][None, :]
        bel = bn_beta[l][None, :]
        if l < _L - 1:
            x, xs = _tc_tail(x, x1, agg2, norm, W[l], bl, gl, bel)
        else:
            y = _tc_tail_last(x, x1, agg2, norm, W[l], bl, gl, bel,
                              mlp_W1, mlp_b1[None, :], mlp_W2, mlp_b2[None, :],
                              mlp_W3, mlp_b3[None, :])
    return y
